# baseline (speedup 1.0000x reference)
import functools

import jax
import jax.numpy as jnp
from jax import lax
from jax.experimental import pallas as pl
from jax.experimental.pallas import tpu as pltpu
from jax.experimental.pallas import tpu_sc as plsc

N_ATOMS = 320000
N_IN = 128
N_HIDDEN = 64
N_MOL = 10000


TILE_M = 32768
N_PAD = 327680
GRID_M = N_PAD // TILE_M
BLOCKS_A = 8
BLOCKS_B = GRID_M - BLOCKS_A


def _mlp_body(x_ref, w1_ref, b1_ref, w2t_ref, b2_ref, y_ref, *, base):
    i = pl.program_id(0) + base
    xt = x_ref[...]
    h = jnp.dot(xt, w1_ref[...], preferred_element_type=jnp.float32)
    h = h + b1_ref[...]
    h = h * (1.0 / (1.0 + jnp.exp(-h)))
    yrow = jax.lax.dot_general(
        w2t_ref[...], h, (((1,), (1,)), ((), ())),
        preferred_element_type=jnp.float32)
    yrow = yrow + b2_ref[...]
    g = i * TILE_M + jax.lax.broadcasted_iota(jnp.int32, (1, TILE_M), 1)
    yrow = jnp.where(g < N_ATOMS, yrow, 0.0)
    y_ref[...] = yrow.reshape(TILE_M)


def _mlp_slab(x, W1, b1, W2, b2, base, nblocks):
    return pl.pallas_call(
        functools.partial(_mlp_body, base=base),
        grid=(nblocks,),
        in_specs=[
            pl.BlockSpec((TILE_M, N_IN), lambda i: (i + base, 0)),
            pl.BlockSpec((N_IN, N_HIDDEN), lambda i: (0, 0)),
            pl.BlockSpec((1, N_HIDDEN), lambda i: (0, 0)),
            pl.BlockSpec((1, N_HIDDEN), lambda i: (0, 0)),
            pl.BlockSpec((1, 1), lambda i: (0, 0)),
        ],
        out_specs=pl.BlockSpec((TILE_M,), lambda i: (i,)),
        out_shape=jax.ShapeDtypeStruct((nblocks * TILE_M,), jnp.float32),
    )(x, W1, b1.reshape(1, N_HIDDEN), W2.reshape(1, N_HIDDEN),
      b2.reshape(1, 1))


NS = 16
ACC = 10240
ACC_W = ACC // NS


def _segsum_slab(ids, yslab, base_atom, n_valid, prev):
    atoms_w = n_valid // NS
    mesh = plsc.VectorSubcoreMesh(
        core_axis_name="c", subcore_axis_name="s", num_cores=1)

    scratch = [
        pltpu.VMEM((atoms_w,), jnp.int32),
        pltpu.VMEM((atoms_w,), jnp.float32),
        pltpu.VMEM((ACC_W,), jnp.float32),
        pltpu.VMEM_SHARED((ACC,), jnp.float32),
        pltpu.SemaphoreType.DMA,
    ]

    def body(ids_hbm, y_hbm, *rest):
        if prev is None:
            out_hbm, idx_v, y_v, stage_v, acc_sh, sem = rest
        else:
            prev_hbm, out_hbm, idx_v, y_v, stage_v, acc_sh, sem = rest
        s = lax.axis_index("s")

        cin = pltpu.async_copy(
            ids_hbm.at[pl.ds(base_atom + s * atoms_w, atoms_w)], idx_v, sem)
        cy = pltpu.async_copy(
            y_hbm.at[pl.ds(s * atoms_w, atoms_w)], y_v, sem)

        if prev is None:
            zeros16 = jnp.zeros((16,), jnp.float32)

            def zbody(i, carry):
                stage_v[pl.ds(i * 16, 16)] = zeros16
                return carry

            lax.fori_loop(0, ACC_W // 16, zbody, 0)
            pltpu.sync_copy(stage_v, acc_sh.at[pl.ds(s * ACC_W, ACC_W)])
        else:
            pltpu.sync_copy(prev_hbm.at[pl.ds(s * ACC_W, ACC_W)],
                            acc_sh.at[pl.ds(s * ACC_W, ACC_W)])
        cin.wait()
        cy.wait()

        plsc.subcore_barrier()

        pltpu.async_copy(y_v, acc_sh.at[idx_v], sem, add=True).wait()

        plsc.subcore_barrier()

        pltpu.sync_copy(acc_sh.at[pl.ds(s * ACC_W, ACC_W)],
                        out_hbm.at[pl.ds(s * ACC_W, ACC_W)])

    run = functools.partial(
        pl.kernel,
        out_type=jax.ShapeDtypeStruct((ACC,), jnp.float32),
        mesh=mesh,
        scratch_types=scratch,
    )(body)
    if prev is None:
        return run(ids, yslab)
    return run(ids, yslab, prev)


def kernel(x, segment_ids, W1, b1, W2, b2):
    ids = (segment_ids if segment_ids.dtype == jnp.int32
           else segment_ids.astype(jnp.int32))
    atoms_a = BLOCKS_A * TILE_M
    y_a = _mlp_slab(x, W1, b1, W2, b2, 0, BLOCKS_A)
    y_b = _mlp_slab(x, W1, b1, W2, b2, BLOCKS_A, BLOCKS_B)
    p_a = _segsum_slab(ids, y_a, 0, atoms_a, None)
    p_b = _segsum_slab(ids, y_b, atoms_a, N_ATOMS - atoms_a, p_a)
    return p_b[:N_MOL]


# speedup vs baseline: 1.0265x; 1.0265x over previous
import functools

import jax
import jax.numpy as jnp
from jax import lax
from jax.experimental import pallas as pl
from jax.experimental.pallas import tpu as pltpu
from jax.experimental.pallas import tpu_sc as plsc

N_ATOMS = 320000
N_IN = 128
N_HIDDEN = 64
N_MOL = 10000


TILE_M = 32768
N_PAD = 327680
GRID_M = N_PAD // TILE_M
BLOCKS_A = 7
BLOCKS_B = GRID_M - BLOCKS_A


def _mlp_body(x_ref, w1_ref, b1_ref, w2t_ref, b2_ref, y_ref, *, base):
    i = pl.program_id(0) + base
    xt = x_ref[...]
    h = jnp.dot(xt, w1_ref[...], preferred_element_type=jnp.float32)
    h = h + b1_ref[...]
    h = h * (1.0 / (1.0 + jnp.exp(-h)))
    yrow = jax.lax.dot_general(
        w2t_ref[...], h, (((1,), (1,)), ((), ())),
        preferred_element_type=jnp.float32)
    yrow = yrow + b2_ref[...]
    g = i * TILE_M + jax.lax.broadcasted_iota(jnp.int32, (1, TILE_M), 1)
    yrow = jnp.where(g < N_ATOMS, yrow, 0.0)
    y_ref[...] = yrow.reshape(TILE_M)


def _mlp_slab(x, W1, b1, W2, b2, base, nblocks):
    return pl.pallas_call(
        functools.partial(_mlp_body, base=base),
        grid=(nblocks,),
        in_specs=[
            pl.BlockSpec((TILE_M, N_IN), lambda i: (i + base, 0)),
            pl.BlockSpec((N_IN, N_HIDDEN), lambda i: (0, 0)),
            pl.BlockSpec((1, N_HIDDEN), lambda i: (0, 0)),
            pl.BlockSpec((1, N_HIDDEN), lambda i: (0, 0)),
            pl.BlockSpec((1, 1), lambda i: (0, 0)),
        ],
        out_specs=pl.BlockSpec((TILE_M,), lambda i: (i,)),
        out_shape=jax.ShapeDtypeStruct((nblocks * TILE_M,), jnp.float32),
    )(x, W1, b1.reshape(1, N_HIDDEN), W2.reshape(1, N_HIDDEN),
      b2.reshape(1, 1))


NS = 16
ACC = 10240
ACC_W = ACC // NS


def _segsum_slab(ids, yslab, base_atom, n_valid, prev):
    atoms_w = n_valid // NS
    mesh = plsc.VectorSubcoreMesh(
        core_axis_name="c", subcore_axis_name="s", num_cores=1)

    scratch = [
        pltpu.VMEM((atoms_w,), jnp.int32),
        pltpu.VMEM((atoms_w,), jnp.float32),
        pltpu.VMEM((ACC_W,), jnp.float32),
        pltpu.VMEM_SHARED((ACC,), jnp.float32),
        pltpu.SemaphoreType.DMA,
    ]

    def body(ids_hbm, y_hbm, *rest):
        if prev is None:
            out_hbm, idx_v, y_v, stage_v, acc_sh, sem = rest
        else:
            prev_hbm, out_hbm, idx_v, y_v, stage_v, acc_sh, sem = rest
        s = lax.axis_index("s")

        cin = pltpu.async_copy(
            ids_hbm.at[pl.ds(base_atom + s * atoms_w, atoms_w)], idx_v, sem)
        cy = pltpu.async_copy(
            y_hbm.at[pl.ds(s * atoms_w, atoms_w)], y_v, sem)

        if prev is None:
            zeros16 = jnp.zeros((16,), jnp.float32)

            def zbody(i, carry):
                stage_v[pl.ds(i * 16, 16)] = zeros16
                return carry

            lax.fori_loop(0, ACC_W // 16, zbody, 0)
            pltpu.sync_copy(stage_v, acc_sh.at[pl.ds(s * ACC_W, ACC_W)])
        else:
            pltpu.sync_copy(prev_hbm.at[pl.ds(s * ACC_W, ACC_W)],
                            acc_sh.at[pl.ds(s * ACC_W, ACC_W)])
        cin.wait()
        cy.wait()

        plsc.subcore_barrier()

        pltpu.async_copy(y_v, acc_sh.at[idx_v], sem, add=True).wait()

        plsc.subcore_barrier()

        pltpu.sync_copy(acc_sh.at[pl.ds(s * ACC_W, ACC_W)],
                        out_hbm.at[pl.ds(s * ACC_W, ACC_W)])

    run = functools.partial(
        pl.kernel,
        out_type=jax.ShapeDtypeStruct((ACC,), jnp.float32),
        mesh=mesh,
        scratch_types=scratch,
    )(body)
    if prev is None:
        return run(ids, yslab)
    return run(ids, yslab, prev)


def kernel(x, segment_ids, W1, b1, W2, b2):
    ids = (segment_ids if segment_ids.dtype == jnp.int32
           else segment_ids.astype(jnp.int32))
    atoms_a = BLOCKS_A * TILE_M
    y_a = _mlp_slab(x, W1, b1, W2, b2, 0, BLOCKS_A)
    y_b = _mlp_slab(x, W1, b1, W2, b2, BLOCKS_A, BLOCKS_B)
    p_a = _segsum_slab(ids, y_a, 0, atoms_a, None)
    p_b = _segsum_slab(ids, y_b, atoms_a, N_ATOMS - atoms_a, p_a)
    return p_b[:N_MOL]
